# baseline (speedup 1.0000x reference)
import jax
import jax.numpy as jnp
from jax.experimental import pallas as pl
from jax.experimental.pallas import tpu as pltpu


def _gating_kernel(nblk, x_ref, w1_ref, b1_ref, w2_ref, b2_ref, o_ref,
                   w1b_ref, w2b_ref, h_ref):
    i = pl.program_id(0)

    @pl.when(i == 0)
    def _cast_weights():
        w1b_ref[...] = w1_ref[...].astype(jnp.bfloat16)
        w2b_ref[...] = w2_ref[...].astype(jnp.bfloat16)

    @pl.when(i < nblk)
    def _stage_a():
        d_model = x_ref.shape[1]
        nk = 4
        ck = d_model // nk
        h = None
        for k in range(nk):
            xb = x_ref[:, k * ck:(k + 1) * ck].astype(jnp.bfloat16)
            p = jnp.dot(xb, w1b_ref[k * ck:(k + 1) * ck, :],
                        preferred_element_type=jnp.float32)
            h = p if h is None else h + p
        h = jnp.maximum(h + b1_ref[...], 0.0)
        slot = jax.lax.rem(i, 2)
        h_ref[pl.ds(slot, 1), :, :] = h.astype(jnp.bfloat16)[None]

    @pl.when(i > 0)
    def _stage_b():
        slot = jax.lax.rem(i - 1, 2)
        hp = h_ref[pl.ds(slot, 1), :, :][0]
        logits = jnp.dot(hp, w2b_ref[...], preferred_element_type=jnp.float32)
        logits = logits + b2_ref[...]
        m = jnp.max(logits, axis=-1, keepdims=True)
        e = jnp.exp(logits - m)
        o_ref[...] = e * (1.0 / jnp.sum(e, axis=-1, keepdims=True))


def kernel(x, W1, b1, W2, b2):
    tokens, d_model = x.shape
    hidden, n_experts = W2.shape
    bt = 512
    nblk = tokens // bt
    b1r = b1.reshape(1, hidden)
    b2r = b2.reshape(1, n_experts)
    import functools
    body = functools.partial(_gating_kernel, nblk)
    return pl.pallas_call(
        body,
        grid=(nblk + 1,),
        in_specs=[
            pl.BlockSpec((bt, d_model), lambda i: (jnp.minimum(i, nblk - 1), 0)),
            pl.BlockSpec((d_model, hidden), lambda i: (0, 0)),
            pl.BlockSpec((1, hidden), lambda i: (0, 0)),
            pl.BlockSpec((hidden, n_experts), lambda i: (0, 0)),
            pl.BlockSpec((1, n_experts), lambda i: (0, 0)),
        ],
        out_specs=pl.BlockSpec((bt, n_experts),
                               lambda i: (jnp.maximum(i - 1, 0), 0)),
        out_shape=jax.ShapeDtypeStruct((tokens, n_experts), jnp.float32),
        scratch_shapes=[
            pltpu.VMEM((d_model, hidden), jnp.bfloat16),
            pltpu.VMEM((hidden, n_experts), jnp.bfloat16),
            pltpu.VMEM((2, bt, hidden), jnp.bfloat16),
        ],
        compiler_params=pltpu.CompilerParams(
            dimension_semantics=("arbitrary",),
        ),
    )(x, W1, b1r, W2, b2r)


# speedup vs baseline: 1.0190x; 1.0190x over previous
import jax
import jax.numpy as jnp
from jax.experimental import pallas as pl
from jax.experimental.pallas import tpu as pltpu


def _gating_kernel(nblk, x_ref, w1_ref, b1_ref, w2_ref, b2_ref, o_ref,
                   w1b_ref, w2b_ref, h_ref):
    i = pl.program_id(0)

    @pl.when(i == 0)
    def _cast_weights():
        w1b_ref[...] = w1_ref[...].astype(jnp.bfloat16)
        w2b_ref[...] = w2_ref[...].astype(jnp.bfloat16)

    slot_b = jax.lax.rem(i + 1, 2)
    hp = h_ref[pl.ds(slot_b, 1), :, :][0]
    logits = jnp.dot(hp, w2b_ref[...], preferred_element_type=jnp.float32)
    logits = logits + b2_ref[...]
    m = jnp.max(logits, axis=-1, keepdims=True)
    e = jnp.exp(logits - m)
    o_ref[...] = e * (1.0 / jnp.sum(e, axis=-1, keepdims=True))

    d_model = x_ref.shape[1]
    nk = 4
    ck = d_model // nk
    h = None
    for k in range(nk):
        xb = x_ref[:, k * ck:(k + 1) * ck].astype(jnp.bfloat16)
        p = jnp.dot(xb, w1b_ref[k * ck:(k + 1) * ck, :],
                    preferred_element_type=jnp.float32)
        h = p if h is None else h + p
    h = jnp.maximum(h + b1_ref[...], 0.0)
    slot_a = jax.lax.rem(i, 2)
    h_ref[pl.ds(slot_a, 1), :, :] = h.astype(jnp.bfloat16)[None]


def kernel(x, W1, b1, W2, b2):
    tokens, d_model = x.shape
    hidden, n_experts = W2.shape
    bt = 512
    nblk = tokens // bt
    b1r = b1.reshape(1, hidden)
    b2r = b2.reshape(1, n_experts)
    import functools
    body = functools.partial(_gating_kernel, nblk)
    return pl.pallas_call(
        body,
        grid=(nblk + 1,),
        in_specs=[
            pl.BlockSpec((bt, d_model), lambda i: (jnp.minimum(i, nblk - 1), 0)),
            pl.BlockSpec((d_model, hidden), lambda i: (0, 0)),
            pl.BlockSpec((1, hidden), lambda i: (0, 0)),
            pl.BlockSpec((hidden, n_experts), lambda i: (0, 0)),
            pl.BlockSpec((1, n_experts), lambda i: (0, 0)),
        ],
        out_specs=pl.BlockSpec((bt, n_experts),
                               lambda i: (jnp.maximum(i - 1, 0), 0)),
        out_shape=jax.ShapeDtypeStruct((tokens, n_experts), jnp.float32),
        scratch_shapes=[
            pltpu.VMEM((d_model, hidden), jnp.bfloat16),
            pltpu.VMEM((hidden, n_experts), jnp.bfloat16),
            pltpu.VMEM((2, bt, hidden), jnp.bfloat16),
        ],
        compiler_params=pltpu.CompilerParams(
            dimension_semantics=("arbitrary",),
        ),
    )(x, W1, b1r, W2, b2r)
